# baseline (speedup 1.0000x reference)
import functools

import jax
import jax.numpy as jnp
from jax import lax
from jax.experimental import pallas as pl
from jax.experimental.pallas import tpu as pltpu
from jax.experimental.pallas import tpu_sc as plsc

F = 26
V = 100000
D = 32
B = 16384
H1 = 512
H2 = 256

NW = 32
NPAIR = F * (D + 1)
PPW = -(-NPAIR // NW)
B1 = 25088
B2 = 50176
B3 = 75264
VT = 99968
BNDS = (0, B1, B2, B3, VT)
WIDS = (B1, B2 - B1, B3 - B2, VT - B3)
BUFW = max(WIDS)
TW = (V - VT) * (D + 1)


def _sc_pool_fn():
    mesh = plsc.VectorSubcoreMesh(core_axis_name="c", subcore_axis_name="s")

    @functools.partial(
        pl.kernel,
        mesh=mesh,
        compiler_params=pltpu.CompilerParams(use_tc_tiling_on_sc=True,
                                             needs_layout_passes=False),
        out_type=jax.ShapeDtypeStruct((NW, 32), jnp.float32),
        scratch_types=[
            pltpu.VMEM((B,), jnp.int32),
            pltpu.VMEM((B + 16,), jnp.int32),
            pltpu.VMEM((BUFW,), jnp.float32),
            pltpu.VMEM((BUFW,), jnp.float32),
            pltpu.VMEM((TW,), jnp.float32),
            pltpu.VMEM((32,), jnp.float32),
            pltpu.SMEM((4,), jnp.int32),
            pltpu.SemaphoreType.DMA,
            pltpu.SemaphoreType.DMA,
        ],
    )
    def sc_kernel(embT_hbm, lin_hbm, xT_hbm, tail_hbm, out_hbm,
                  x_v, part_v, buf0, buf1, tail_v, out_v, cnt_s, sem0, sem1):
        wid = lax.axis_index("s") * 2 + lax.axis_index("c")
        bufs = (buf0, buf1)
        sems = (sem0, sem1)
        iota16 = lax.iota(jnp.int32, 16)
        out_v[pl.ds(0, 16)] = jnp.zeros((16,), jnp.float32)
        out_v[pl.ds(16, 16)] = jnp.zeros((16,), jnp.float32)

        def pair_of(j):
            p = jnp.minimum(wid * PPW + j, NPAIR - 1)
            return p // (D + 1), p % (D + 1)

        def issue(j, s):
            f, k = pair_of(j)
            dst = bufs[s % 2].at[pl.ds(0, WIDS[s])]
            sl = pl.ds(BNDS[s], WIDS[s])

            @pl.when(k < D)
            def _():
                pltpu.async_copy(embT_hbm.at[f, k, sl], dst, sems[s % 2])

            @pl.when(k == D)
            def _():
                pltpu.async_copy(lin_hbm.at[f, 0, sl], dst, sems[s % 2])

        def wait(s):
            pltpu.make_async_copy(
                embT_hbm.at[0, 0, pl.ds(BNDS[s], WIDS[s])],
                bufs[s % 2].at[pl.ds(0, WIDS[s])], sems[s % 2]).wait()

        def partition(f):
            pltpu.sync_copy(xT_hbm.at[f], x_v)
            pltpu.sync_copy(tail_hbm.at[f], tail_v)

            def p1(i, c):
                n0, n01, n012, n0123 = c
                for u in range(2):
                    v = x_v[pl.ds(i * 32 + u * 16, 16)]
                    n0 = n0 + jnp.sum((v < B1).astype(jnp.int32))
                    n01 = n01 + jnp.sum((v < B2).astype(jnp.int32))
                    n012 = n012 + jnp.sum((v < B3).astype(jnp.int32))
                    n0123 = n0123 + jnp.sum((v < VT).astype(jnp.int32))
                return n0, n01, n012, n0123

            z = jnp.int32(0)
            n0, n01, n012, n0123 = lax.fori_loop(0, B // 32, p1, (z, z, z, z))
            cnt_s[0] = n0
            cnt_s[1] = n01
            cnt_s[2] = n012
            cnt_s[3] = n0123

            def p2(i, offs):
                o0, o1, o2, o3, o4 = offs
                v = x_v[pl.ds(i * 16, 16)]
                l1 = v < B1
                l2 = v < B2
                l3 = v < B3
                l4 = v < VT
                m0 = l1
                m1 = jnp.logical_and(l2, jnp.logical_not(l1))
                m2 = jnp.logical_and(l3, jnp.logical_not(l2))
                m3 = jnp.logical_and(l4, jnp.logical_not(l3))
                m4 = jnp.logical_not(l4)
                plsc.store_compressed(part_v.at[pl.ds(o0, 16)], v, mask=m0)
                plsc.store_compressed(part_v.at[pl.ds(o1, 16)], v, mask=m1)
                plsc.store_compressed(part_v.at[pl.ds(o2, 16)], v, mask=m2)
                plsc.store_compressed(part_v.at[pl.ds(o3, 16)], v, mask=m3)
                plsc.store_compressed(part_v.at[pl.ds(o4, 16)], v, mask=m4)
                return (o0 + jnp.sum(m0.astype(jnp.int32)),
                        o1 + jnp.sum(m1.astype(jnp.int32)),
                        o2 + jnp.sum(m2.astype(jnp.int32)),
                        o3 + jnp.sum(m3.astype(jnp.int32)),
                        o4 + jnp.sum(m4.astype(jnp.int32)))

            lax.fori_loop(0, B // 16, p2, (z, n0, n01, n012, n0123))

        issue(0, 0)

        def row_body(j, prev_f):
            f, k = pair_of(j)

            @pl.when(f != prev_f)
            def _():
                partition(f)

            acc = jnp.zeros((16,), jnp.float32)
            for s in range(4):
                if s < 3:
                    issue(j, s + 1)
                else:
                    issue(j + 1, 0)
                wait(s)
                n0 = cnt_s[0]
                n01 = cnt_s[1]
                n012 = cnt_s[2]
                n0123 = cnt_s[3]
                off = (jnp.int32(0), n0, n01, n012)[s]
                end = (n0, n01, n012, n0123)[s]
                n = end - off
                buf = bufs[s % 2]

                def g4(i, a, off=off, buf=buf, s=s):
                    base = off + i * 64
                    for u in range(4):
                        idxs = part_v[pl.ds(base + u * 16, 16)] - BNDS[s]
                        a = a + plsc.load_gather(buf, [idxs])
                    return a

                def g1(i, a, off=off, buf=buf, s=s):
                    idxs = part_v[pl.ds(off + i * 16, 16)] - BNDS[s]
                    return a + plsc.load_gather(buf, [idxs])

                acc = lax.fori_loop(0, n // 64, g4, acc)
                acc = lax.fori_loop((n // 64) * 4, n // 16, g1, acc)
                rem = n % 16
                m = iota16 < rem
                tidx = part_v[pl.ds(off + (n // 16) * 16, 16)] - BNDS[s]
                tidx = jnp.minimum(jnp.maximum(tidx, 0), WIDS[s] - 1)
                tv = plsc.load_gather(buf, [tidx], mask=m)
                acc = acc + jnp.where(m, tv, 0.0)

            toff = cnt_s[3]
            tn = B - toff

            def tg(i, a):
                idxs = (part_v[pl.ds(toff + i * 16, 16)] - VT) * (D + 1) + k
                return a + plsc.load_gather(tail_v, [idxs])

            acc = lax.fori_loop(0, tn // 16, tg, acc)
            trem = tn % 16
            tm = iota16 < trem
            txi = (part_v[pl.ds(toff + (tn // 16) * 16, 16)] - VT) * (D + 1) + k
            txi = jnp.minimum(jnp.maximum(txi, 0), TW - 1)
            ttv = plsc.load_gather(tail_v, [txi], mask=tm)
            acc = acc + jnp.where(tm, ttv, 0.0)

            valid = wid * PPW + j < NPAIR
            sv = jnp.sum(acc)

            @pl.when(valid)
            def _():
                plsc.store_scatter(
                    out_v, [jnp.full((16,), j, jnp.int32)],
                    jnp.full((16,), sv, jnp.float32),
                    mask=iota16 == 0)

            return f

        lax.fori_loop(0, PPW, row_body, jnp.int32(-1))
        wait(0)
        pltpu.sync_copy(out_v, out_hbm.at[wid])

    return sc_kernel


def _tc_head(s_flat, s3, lin_s, biasr, W1, g1r, be1r, W2, g2r, be2r,
             w3r, b3r):
    def tc_kernel(pf_ref, p3_ref, pl_ref, bias_ref, W1_ref, g1_ref, be1_ref,
                  W2_ref, g2_ref, be2_ref, w3_ref, b3_ref, out_ref):
        Bf = jnp.float32(B)
        s_row = pf_ref[...]
        s3v = p3_ref[...]
        s_lin = jnp.sum(pl_ref[...]).reshape(1, 1)
        colsum = jnp.sum(s3v, axis=0, keepdims=True)
        inner = 0.5 * (jnp.sum(colsum * colsum).reshape(1, 1)
                       - jnp.sum(s3v * s3v).reshape(1, 1))

        d1 = jnp.dot(s_row, W1_ref[...],
                     preferred_element_type=jnp.float32)
        inv1 = lax.rsqrt(d1 * d1 * ((Bf - 1.0) / (Bf * Bf)) + 1e-5)
        a_a = jnp.maximum((-d1 / Bf) * inv1 * g1_ref[...] + be1_ref[...], 0.0)
        a_b = jnp.maximum((d1 * ((Bf - 1.0) / Bf)) * inv1 * g1_ref[...]
                          + be1_ref[...], 0.0)
        a = jnp.concatenate([a_a, a_b], axis=0)

        h2 = jnp.dot(a, W2_ref[...],
                     preferred_element_type=jnp.float32)
        d2 = h2[1:2, :] - h2[0:1, :]
        inv2 = lax.rsqrt(d2 * d2 * ((Bf - 1.0) / (Bf * Bf)) + 1e-5)
        r_a = jnp.maximum((-d2 / Bf) * inv2 * g2_ref[...] + be2_ref[...], 0.0)
        r_b = jnp.maximum((d2 * ((Bf - 1.0) / Bf)) * inv2 * g2_ref[...]
                          + be2_ref[...], 0.0)
        r = jnp.concatenate([r_a, r_b], axis=0)

        m = jnp.sum(r * w3_ref[...], axis=1, keepdims=True) + b3_ref[...]
        la = bias_ref[...] + m[0:1, :]
        lb = bias_ref[...] + s_lin + inner + m[1:2, :]
        sa = 1.0 / (1.0 + jnp.exp(-la))
        sb = 1.0 / (1.0 + jnp.exp(-lb))
        lane = lax.broadcasted_iota(jnp.int32, (1, B), 1)
        out_ref[...] = jnp.where(lane == B - 1, sb, sa)

    return pl.pallas_call(
        tc_kernel,
        out_shape=jax.ShapeDtypeStruct((1, B), jnp.float32),
    )(s_flat, s3, lin_s, biasr, W1, g1r, be1r, W2, g2r, be2r, w3r, b3r)


def kernel(x, emb_table, lin_table, bias, W1, b1, g1, be1, W2, b2, g2, be2,
           W3, b3):
    del b1, b2
    embT = jnp.transpose(emb_table, (0, 2, 1))
    lin3 = jnp.transpose(lin_table, (0, 2, 1))
    xT = x.astype(jnp.int32).T
    tail = jnp.concatenate(
        [emb_table[:, VT:, :], lin_table[:, VT:, :]], axis=2).reshape(F, TW)

    out = _sc_pool_fn()(embT, lin3, xT, tail)
    vals = out[:, :PPW].reshape(NW * PPW)[:NPAIR].reshape(F, D + 1)
    s3 = vals[:, :D]
    lin_s = vals[:, D].reshape(1, F)
    out2 = _tc_head(
        s3.reshape(1, F * D), s3, lin_s,
        bias.reshape(1, 1), W1, g1.reshape(1, H1), be1.reshape(1, H1),
        W2, g2.reshape(1, H2), be2.reshape(1, H2),
        W3.reshape(1, H2), b3.reshape(1, 1))
    return out2.reshape(B)


# speedup vs baseline: 1.0868x; 1.0868x over previous
import functools

import jax
import jax.numpy as jnp
from jax import lax
from jax.experimental import pallas as pl
from jax.experimental.pallas import tpu as pltpu
from jax.experimental.pallas import tpu_sc as plsc

F = 26
V = 100000
D = 32
B = 16384
H1 = 512
H2 = 256

NW = 32
NPAIR = F * (D + 1)
PPW = -(-NPAIR // NW)
GU = 4


def _sc_pool_fn():
    mesh = plsc.VectorSubcoreMesh(core_axis_name="c", subcore_axis_name="s")

    @functools.partial(
        pl.kernel,
        mesh=mesh,
        compiler_params=pltpu.CompilerParams(use_tc_tiling_on_sc=True,
                                             needs_layout_passes=False),
        out_type=jax.ShapeDtypeStruct((NW, 32), jnp.float32),
        scratch_types=[
            pltpu.VMEM((B,), jnp.int32),
            pltpu.VMEM((V,), jnp.float32),
            pltpu.VMEM((32,), jnp.float32),
        ],
    )
    def sc_kernel(embT_hbm, lin_hbm, xT_hbm, out_hbm, x_v, row_v, out_v):
        wid = lax.axis_index("s") * 2 + lax.axis_index("c")
        out_v[pl.ds(0, 16)] = jnp.zeros((16,), jnp.float32)
        out_v[pl.ds(16, 16)] = jnp.zeros((16,), jnp.float32)

        def pair_body(j, prev_f):
            p = wid * PPW + j
            valid = p < NPAIR
            pc = jnp.where(valid, p, 0)
            f = pc // (D + 1)
            k = pc % (D + 1)

            @pl.when(valid)
            def _():
                @pl.when(f != prev_f)
                def _():
                    pltpu.sync_copy(xT_hbm.at[f], x_v)

                @pl.when(k < D)
                def _():
                    pltpu.sync_copy(embT_hbm.at[f, k], row_v)

                @pl.when(k == D)
                def _():
                    pltpu.sync_copy(lin_hbm.at[f, 0], row_v)

                def gbody(i, acc):
                    for u in range(GU):
                        idxs = x_v[pl.ds(i * (16 * GU) + u * 16, 16)]
                        acc = acc + plsc.load_gather(row_v, [idxs])
                    return acc

                acc = lax.fori_loop(0, B // (16 * GU), gbody,
                                    jnp.zeros((16,), jnp.float32))
                s = jnp.sum(acc)
                plsc.store_scatter(
                    out_v, [jnp.full((16,), j, jnp.int32)],
                    jnp.full((16,), s, jnp.float32),
                    mask=lax.iota(jnp.int32, 16) == 0)

            return jnp.where(valid, f, prev_f)

        lax.fori_loop(0, PPW, pair_body, jnp.int32(-1))
        pltpu.sync_copy(out_v, out_hbm.at[wid])

    return sc_kernel


def _tc_head(s_flat, s3, lin_s, biasr, W1, g1r, be1r, W2, g2r, be2r,
             w3r, b3r):
    def tc_kernel(pf_ref, p3_ref, pl_ref, bias_ref, W1_ref, g1_ref, be1_ref,
                  W2_ref, g2_ref, be2_ref, w3_ref, b3_ref, out_ref):
        Bf = jnp.float32(B)
        s_row = pf_ref[...]
        s3v = p3_ref[...]
        s_lin = jnp.sum(pl_ref[...]).reshape(1, 1)
        colsum = jnp.sum(s3v, axis=0, keepdims=True)
        inner = 0.5 * (jnp.sum(colsum * colsum).reshape(1, 1)
                       - jnp.sum(s3v * s3v).reshape(1, 1))

        d1 = jnp.dot(s_row, W1_ref[...],
                     preferred_element_type=jnp.float32)
        inv1 = lax.rsqrt(d1 * d1 * ((Bf - 1.0) / (Bf * Bf)) + 1e-5)
        a_a = jnp.maximum((-d1 / Bf) * inv1 * g1_ref[...] + be1_ref[...], 0.0)
        a_b = jnp.maximum((d1 * ((Bf - 1.0) / Bf)) * inv1 * g1_ref[...]
                          + be1_ref[...], 0.0)
        a = jnp.concatenate([a_a, a_b], axis=0)

        h2 = jnp.dot(a, W2_ref[...],
                     preferred_element_type=jnp.float32)
        d2 = h2[1:2, :] - h2[0:1, :]
        inv2 = lax.rsqrt(d2 * d2 * ((Bf - 1.0) / (Bf * Bf)) + 1e-5)
        r_a = jnp.maximum((-d2 / Bf) * inv2 * g2_ref[...] + be2_ref[...], 0.0)
        r_b = jnp.maximum((d2 * ((Bf - 1.0) / Bf)) * inv2 * g2_ref[...]
                          + be2_ref[...], 0.0)
        r = jnp.concatenate([r_a, r_b], axis=0)

        m = jnp.sum(r * w3_ref[...], axis=1, keepdims=True) + b3_ref[...]
        la = bias_ref[...] + m[0:1, :]
        lb = bias_ref[...] + s_lin + inner + m[1:2, :]
        sa = 1.0 / (1.0 + jnp.exp(-la))
        sb = 1.0 / (1.0 + jnp.exp(-lb))
        lane = lax.broadcasted_iota(jnp.int32, (1, B), 1)
        out_ref[...] = jnp.where(lane == B - 1, sb, sa)

    return pl.pallas_call(
        tc_kernel,
        out_shape=jax.ShapeDtypeStruct((1, B), jnp.float32),
    )(s_flat, s3, lin_s, biasr, W1, g1r, be1r, W2, g2r, be2r, w3r, b3r)


def kernel(x, emb_table, lin_table, bias, W1, b1, g1, be1, W2, b2, g2, be2,
           W3, b3):
    del b1, b2
    embT = jnp.transpose(emb_table, (0, 2, 1))
    lin3 = jnp.transpose(lin_table, (0, 2, 1))
    xT = x.astype(jnp.int32).T

    out = _sc_pool_fn()(embT, lin3, xT)
    vals = out[:, :PPW].reshape(NW * PPW)[:NPAIR].reshape(F, D + 1)
    s3 = vals[:, :D]
    lin_s = vals[:, D].reshape(1, F)
    out2 = _tc_head(
        s3.reshape(1, F * D), s3, lin_s,
        bias.reshape(1, 1), W1, g1.reshape(1, H1), be1.reshape(1, H1),
        W2, g2.reshape(1, H2), be2.reshape(1, H2),
        W3.reshape(1, H2), b3.reshape(1, 1))
    return out2.reshape(B)
